# baseline (speedup 1.0000x reference)
import jax
import jax.numpy as jnp
from jax.experimental import pallas as pl

_B, _N, _D, _DFF, _E = 1024, 5, 256, 512, 8
_TEMP = 0.07
_BB = 256
_TT = _BB * _N


def _top2_gates(logits):
    ii = jax.lax.broadcasted_iota(jnp.int32, logits.shape, 1)
    m1 = jnp.max(logits, axis=1, keepdims=True)
    i1 = jnp.min(jnp.where(logits == m1, ii, _E), axis=1, keepdims=True)
    oh1 = ii == i1
    l2 = jnp.where(oh1, -jnp.inf, logits)
    m2 = jnp.max(l2, axis=1, keepdims=True)
    i2 = jnp.min(jnp.where(l2 == m2, ii, _E), axis=1, keepdims=True)
    oh2 = ii == i2
    e21 = jnp.exp(m2 - m1)
    g1 = 1.0 / (1.0 + e21)
    g2 = e21 * g1
    return jnp.where(oh1, g1, 0.0) + jnp.where(oh2, g2, 0.0)


def _mm(a, b):
    return jax.lax.dot_general(a, b, (((1,), (0,)), ((), ())),
                               preferred_element_type=jnp.float32)


def _moe(x, Wr, W1, W2):
    logits = _mm(x, Wr)
    fg = _top2_gates(logits)
    xb = x.astype(jnp.bfloat16)
    acc = jnp.zeros((x.shape[0], _D), jnp.float32)
    for e in range(_E):
        h = jax.nn.gelu(_mm(xb, W1[e]))
        acc = acc + fg[:, e:e + 1] * _mm(h.astype(jnp.bfloat16), W2[e])
    return acc


def _l2n(x):
    n = jnp.sqrt(jnp.sum(x * x, axis=-1, keepdims=True))
    return x / jnp.maximum(n, 1e-12)


def _body(q_ref, items_ref, xt_ref, xl_ref,
          wrt_ref, w1t_ref, w2t_ref, wrl_ref, w1l_ref, w2l_ref, out_ref):
    i = pl.program_id(0)

    @pl.when(i == 0)
    def _():
        out_ref[...] = jnp.zeros((1, 1), jnp.float32)

    xt = xt_ref[...].reshape(_TT, _D)
    xl = xl_ref[...].reshape(_TT, _D)
    yt = _moe(xt, wrt_ref[...], w1t_ref, w2t_ref).reshape(_BB, _N, _D)
    yl = _moe(xl, wrl_ref[...], w1l_ref, w2l_ref).reshape(_BB, _N, _D)

    items = items_ref[...]
    q = q_ref[...]
    pos = jnp.concatenate(
        [q[:, None, :], items[:, 0:1], yl[:, 0:1], yt[:, 0:1]], axis=1)
    neg = jnp.concatenate([items[:, 1:], yl[:, 1:], yt[:, 1:]], axis=1)
    pos = _l2n(pos)
    neg = _l2n(neg)
    allf = jnp.concatenate([pos, neg], axis=1)

    iota_a = jax.lax.broadcasted_iota(jnp.int32, (_BB, 4), 1)
    exp_pos = jnp.zeros((_BB, 4), jnp.float32)
    exp_neg = jnp.zeros((_BB, 4), jnp.float32)
    for k in range(16):
        s = jnp.sum(pos * allf[:, k:k + 1, :], axis=2)
        ek = jnp.exp(s / _TEMP)
        if k < 4:
            exp_pos = exp_pos + jnp.where(iota_a == k, 0.0, ek)
        else:
            exp_neg = exp_neg + ek
    ratio = exp_pos / (exp_pos + exp_neg + 1e-8)
    ratio = jnp.where(jnp.isnan(ratio), 0.0, ratio)
    out_ref[...] += -jnp.sum(jnp.log(ratio)).reshape(1, 1)


def kernel(query_emb, items_emb, item_text_tokens, item_llm_tokens,
           Wr_text, W1_text, W2_text, Wr_llm, W1_llm, W2_llm):
    grid = _B // _BB
    tok3 = pl.BlockSpec((_BB, _N, _D), lambda i: (i, 0, 0))
    full = lambda shape: pl.BlockSpec(shape, lambda i: (0,) * len(shape))
    out = pl.pallas_call(
        _body,
        grid=(grid,),
        in_specs=[
            pl.BlockSpec((_BB, _D), lambda i: (i, 0)),
            tok3, tok3, tok3,
            full((_D, _E)), full((_E, _D, _DFF)), full((_E, _DFF, _D)),
            full((_D, _E)), full((_E, _D, _DFF)), full((_E, _DFF, _D)),
        ],
        out_specs=pl.BlockSpec((1, 1), lambda i: (0, 0)),
        out_shape=jax.ShapeDtypeStruct((1, 1), jnp.float32),
    )(query_emb, items_emb, item_text_tokens, item_llm_tokens,
      Wr_text, W1_text.astype(jnp.bfloat16), W2_text.astype(jnp.bfloat16),
      Wr_llm, W1_llm.astype(jnp.bfloat16), W2_llm.astype(jnp.bfloat16))
    return out[0, 0] / (_B * 4)


# speedup vs baseline: 1.3046x; 1.3046x over previous
import jax
import jax.numpy as jnp
from jax.experimental import pallas as pl

_B, _N, _D, _DFF, _E = 1024, 5, 256, 512, 8
_TEMP = 0.07
_BB = 256
_TT = _BB * _N


def _top2_gates(logits):
    ii = jax.lax.broadcasted_iota(jnp.int32, logits.shape, 1)
    m1 = jnp.max(logits, axis=1, keepdims=True)
    i1 = jnp.min(jnp.where(logits == m1, ii, _E), axis=1, keepdims=True)
    oh1 = ii == i1
    l2 = jnp.where(oh1, -jnp.inf, logits)
    m2 = jnp.max(l2, axis=1, keepdims=True)
    i2 = jnp.min(jnp.where(l2 == m2, ii, _E), axis=1, keepdims=True)
    oh2 = ii == i2
    e21 = jnp.exp(m2 - m1)
    g1 = 1.0 / (1.0 + e21)
    g2 = e21 * g1
    return jnp.where(oh1, g1, 0.0) + jnp.where(oh2, g2, 0.0)


def _mm(a, b):
    return jax.lax.dot_general(a, b, (((1,), (0,)), ((), ())),
                               preferred_element_type=jnp.float32)


def _moe(x, Wr, W1, W2):
    logits = _mm(x, Wr)
    fg = _top2_gates(logits).astype(jnp.bfloat16)
    xb = x.astype(jnp.bfloat16)
    acc = jnp.zeros((x.shape[0], _D), jnp.bfloat16)
    for e in range(_E):
        h = jax.nn.gelu(_mm(xb, W1[e]).astype(jnp.bfloat16))
        acc = acc + fg[:, e:e + 1] * _mm(h, W2[e]).astype(jnp.bfloat16)
    return acc.astype(jnp.float32)


def _l2n(x):
    n = jnp.sqrt(jnp.sum(x * x, axis=-1, keepdims=True))
    return x / jnp.maximum(n, 1e-12)


def _body(q_ref, items_ref, xt_ref, xl_ref,
          wrt_ref, w1t_ref, w2t_ref, wrl_ref, w1l_ref, w2l_ref, out_ref):
    i = pl.program_id(0)

    @pl.when(i == 0)
    def _():
        out_ref[...] = jnp.zeros((1, 1), jnp.float32)

    xt = xt_ref[...].reshape(_TT, _D)
    xl = xl_ref[...].reshape(_TT, _D)
    yt = _moe(xt, wrt_ref[...], w1t_ref, w2t_ref).reshape(_BB, _N, _D)
    yl = _moe(xl, wrl_ref[...], w1l_ref, w2l_ref).reshape(_BB, _N, _D)

    items = items_ref[...]
    q = q_ref[...]
    pos = jnp.concatenate(
        [q[:, None, :], items[:, 0:1], yl[:, 0:1], yt[:, 0:1]], axis=1)
    neg = jnp.concatenate([items[:, 1:], yl[:, 1:], yt[:, 1:]], axis=1)
    pos = _l2n(pos)
    neg = _l2n(neg)
    allf = jnp.concatenate([pos, neg], axis=1)

    iota_a = jax.lax.broadcasted_iota(jnp.int32, (_BB, 4), 1)
    exp_pos = jnp.zeros((_BB, 4), jnp.float32)
    exp_neg = jnp.zeros((_BB, 4), jnp.float32)
    for k in range(16):
        s = jnp.sum(pos * allf[:, k:k + 1, :], axis=2)
        ek = jnp.exp(s / _TEMP)
        if k < 4:
            exp_pos = exp_pos + jnp.where(iota_a == k, 0.0, ek)
        else:
            exp_neg = exp_neg + ek
    ratio = exp_pos / (exp_pos + exp_neg + 1e-8)
    ratio = jnp.where(jnp.isnan(ratio), 0.0, ratio)
    out_ref[...] += -jnp.sum(jnp.log(ratio)).reshape(1, 1)


def kernel(query_emb, items_emb, item_text_tokens, item_llm_tokens,
           Wr_text, W1_text, W2_text, Wr_llm, W1_llm, W2_llm):
    grid = _B // _BB
    tok3 = pl.BlockSpec((_BB, _N, _D), lambda i: (i, 0, 0))
    full = lambda shape: pl.BlockSpec(shape, lambda i: (0,) * len(shape))
    out = pl.pallas_call(
        _body,
        grid=(grid,),
        in_specs=[
            pl.BlockSpec((_BB, _D), lambda i: (i, 0)),
            tok3, tok3, tok3,
            full((_D, _E)), full((_E, _D, _DFF)), full((_E, _DFF, _D)),
            full((_D, _E)), full((_E, _D, _DFF)), full((_E, _DFF, _D)),
        ],
        out_specs=pl.BlockSpec((1, 1), lambda i: (0, 0)),
        out_shape=jax.ShapeDtypeStruct((1, 1), jnp.float32),
    )(query_emb, items_emb, item_text_tokens, item_llm_tokens,
      Wr_text, W1_text.astype(jnp.bfloat16), W2_text.astype(jnp.bfloat16),
      Wr_llm, W1_llm.astype(jnp.bfloat16), W2_llm.astype(jnp.bfloat16))
    return out[0, 0] / (_B * 4)
